# baseline (speedup 1.0000x reference)
import jax
import jax.numpy as jnp
from jax.experimental import pallas as pl


def _copy_body(in_ref, out_ref):
    out_ref[...] = in_ref[...]


def kernel(x, embed_weight):
    del embed_weight
    b, s, d = x.shape
    rows = b * s
    x2 = x.reshape(rows, d)
    block_rows = 512
    out = pl.pallas_call(
        _copy_body,
        out_shape=jax.ShapeDtypeStruct((rows, d), x.dtype),
        grid=(rows // block_rows,),
        in_specs=[pl.BlockSpec((block_rows, d), lambda i: (i, 0))],
        out_specs=pl.BlockSpec((block_rows, d), lambda i: (i, 0)),
    )(x2)
    return out.reshape(b, s, d)
